# Initial kernel scaffold; baseline (speedup 1.0000x reference)
import jax
import jax.numpy as jnp
from jax.experimental import pallas as pl


def kernel(x, edge_index, W1_l, b1, W1_r, W2_l, b2, W2_r):
    raise NotImplementedError("write your pallas kernel here")


# speedup vs baseline: 4.7653x; 4.7653x over previous
import functools

import jax
import jax.numpy as jnp
from jax import lax
from jax.experimental import pallas as pl
from jax.experimental.pallas import tpu as pltpu
from jax.experimental.pallas import tpu_sc as plsc

N = 10000
E = 160000
D = 256
HD = 128
NC = 2
NS = 16
EPT = E // NS
K = 80
NCHUNK = EPT // K
NPAD = 10240
RPT = NPAD // NS
ZROWS = 32
DEG_PAD = 10240
DPT = DEG_PAD // NS


def _sc_agg_body(x_hbm, src_hbm, dst_hbm, summed_hbm, deg_hbm,
                 src_v, dst_v, rows_v, ones_v, zbuf, zdeg, acc, degacc, sem):
    c = lax.axis_index("c")
    s = lax.axis_index("s")
    zero16 = jnp.zeros((16,), jnp.float32)
    one16 = jnp.ones((16,), jnp.float32)

    def _zrow(t, carry):
        i = t // (HD // 16)
        k = t % (HD // 16)
        zbuf[i, pl.ds(k * 16, 16)] = zero16
        return carry
    lax.fori_loop(0, ZROWS * (HD // 16), _zrow, None)

    def _zdeg(t, carry):
        zdeg[pl.ds(t * 16, 16)] = zero16
        return carry
    lax.fori_loop(0, DPT // 16, _zdeg, None)

    def _ones(t, carry):
        ones_v[pl.ds(t * 16, 16)] = one16
        return carry
    lax.fori_loop(0, K // 16, _ones, None)

    for k in range(RPT // ZROWS):
        pltpu.sync_copy(zbuf, acc.at[pl.ds(s * RPT + k * ZROWS, ZROWS)])

    @pl.when(c == 0)
    def _():
        pltpu.sync_copy(zdeg, degacc.at[pl.ds(s * DPT, DPT)])

    plsc.subcore_barrier()

    pltpu.sync_copy(src_hbm.at[c, s], src_v)
    pltpu.sync_copy(dst_hbm.at[s], dst_v)

    def _chunk(j, carry):
        pltpu.async_copy(x_hbm.at[src_v.at[j]], rows_v, sem).wait()
        pltpu.sync_copy(rows_v, acc.at[dst_v.at[j]], add=True)

        @pl.when(c == 0)
        def _():
            pltpu.sync_copy(ones_v, degacc.at[dst_v.at[j]], add=True)
        return carry
    lax.fori_loop(0, NCHUNK, _chunk, None)

    plsc.subcore_barrier()

    pltpu.sync_copy(acc.at[pl.ds(s * RPT, RPT)], summed_hbm.at[c, pl.ds(s * RPT, RPT)])

    @pl.when(c == 0)
    def _():
        pltpu.sync_copy(degacc.at[pl.ds(s * DPT, DPT)], deg_hbm.at[pl.ds(s * DPT, DPT)])


_sc_aggregate = functools.partial(
    pl.kernel,
    out_type=[jax.ShapeDtypeStruct((NC, NPAD, HD), jnp.float32),
              jax.ShapeDtypeStruct((DEG_PAD,), jnp.float32)],
    mesh=plsc.VectorSubcoreMesh(core_axis_name="c", subcore_axis_name="s"),
    scratch_types=[
        pltpu.VMEM((NCHUNK, K), jnp.int32),
        pltpu.VMEM((NCHUNK, K), jnp.int32),
        pltpu.VMEM((K, HD), jnp.float32),
        pltpu.VMEM((K,), jnp.float32),
        pltpu.VMEM((ZROWS, HD), jnp.float32),
        pltpu.VMEM((DPT,), jnp.float32),
        pltpu.VMEM_SHARED((NPAD, HD), jnp.float32),
        pltpu.VMEM_SHARED((DEG_PAD,), jnp.float32),
        pltpu.SemaphoreType.DMA,
    ],
)(_sc_agg_body)


def _tc_layer_body(relu, in_split, out_split,
                   sref, dref, xref, wlref, bref, wrref, oref):
    agg = jnp.concatenate([sref[0], sref[1]], axis=-1)
    rec = 1.0 / jnp.maximum(dref[...], 1.0)
    agg = agg * rec
    if in_split:
        xx = jnp.concatenate([xref[0], xref[1]], axis=-1)
    else:
        xx = xref[...]
    o = (jnp.dot(agg, wlref[...], preferred_element_type=jnp.float32)
         + bref[...]
         + jnp.dot(xx, wrref[...], preferred_element_type=jnp.float32))
    if relu:
        o = jnp.maximum(o, 0.0)
    if out_split:
        oref[0] = o[:, :HD]
        oref[1] = o[:, HD:]
    else:
        oref[...] = o


def _tc_layer(summed, deg_col, xin, W_l, b, W_r, *, relu, in_split, out_split):
    BM = 1000
    grid = (N // BM,)
    split_spec = pl.BlockSpec((NC, BM, HD), lambda i: (0, i, 0))
    dense_spec = pl.BlockSpec((BM, D), lambda i: (i, 0))
    in_specs = [
        split_spec,
        pl.BlockSpec((BM, 1), lambda i: (i, 0)),
        split_spec if in_split else dense_spec,
        pl.BlockSpec((D, D), lambda i: (0, 0)),
        pl.BlockSpec((1, D), lambda i: (0, 0)),
        pl.BlockSpec((D, D), lambda i: (0, 0)),
    ]
    if out_split:
        out_spec = split_spec
        out_shape = jax.ShapeDtypeStruct((NC, N, HD), jnp.float32)
    else:
        out_spec = dense_spec
        out_shape = jax.ShapeDtypeStruct((N, D), jnp.float32)
    return pl.pallas_call(
        functools.partial(_tc_layer_body, relu, in_split, out_split),
        grid=grid,
        in_specs=in_specs,
        out_specs=out_spec,
        out_shape=out_shape,
    )(summed, deg_col, xin, W_l, b.reshape(1, D), W_r)


def kernel(x, edge_index, W1_l, b1, W1_r, W2_l, b2, W2_r):
    src = edge_index[0].astype(jnp.int32)
    dst = edge_index[1].astype(jnp.int32)
    sr = src.reshape(NS, NCHUNK, K)
    src_idx = jnp.stack([sr, sr + N])
    dst_idx = dst.reshape(NS, NCHUNK, K)
    x_flat = jnp.concatenate([x[:, :HD], x[:, HD:]], axis=0)

    summed1, deg_pad = _sc_aggregate(x_flat, src_idx, dst_idx)
    deg_col = deg_pad[:N].reshape(N, 1)
    h_split = _tc_layer(summed1, deg_col, x, W1_l, b1, W1_r,
                        relu=True, in_split=False, out_split=True)
    summed2, _ = _sc_aggregate(h_split.reshape(NC * N, HD), src_idx, dst_idx)
    out = _tc_layer(summed2, deg_col, h_split, W2_l, b2, W2_r,
                    relu=False, in_split=True, out_split=False)
    return out
